# baseline (speedup 1.0000x reference)
import functools
import math

import jax
import jax.numpy as jnp
from jax import lax
from jax.experimental import pallas as pl
from jax.experimental.pallas import tpu as pltpu
from jax.experimental.pallas import tpu_sc as plsc

N_VOCAB = 100000
HIDDEN = 64
BATCH = 4096
SEQ = 200
SCALE = math.sqrt(HIDDEN)

NW = 32
NBG = BATCH // 128
NUNIT = SEQ * NBG
PER_W = NUNIT // NW
TBLK = 1024


def _scale_body(t_ref, o_ref):
    o_ref[:, 0:HIDDEN] = t_ref[...].T * SCALE


def _scale_table(emb_weight):
    embT = emb_weight.T
    scaled = pl.pallas_call(
        _scale_body,
        grid=((N_VOCAB + TBLK - 1) // TBLK,),
        in_specs=[pl.BlockSpec((HIDDEN, TBLK), lambda i: (0, i))],
        out_specs=pl.BlockSpec((TBLK, 2 * HIDDEN), lambda i: (i, 0)),
        out_shape=jax.ShapeDtypeStruct((N_VOCAB, 2 * HIDDEN), jnp.float32),
    )(embT)
    return scaled.reshape(2 * N_VOCAB, HIDDEN)


NB = 4


def _gather_kernel(x_hbm, table_hbm, out_hbm, idx_v, *bufs):
    rows = bufs[0:NB]
    trs = bufs[NB:2 * NB]
    gsems = bufs[2 * NB:3 * NB]
    osems = bufs[3 * NB:4 * NB]
    c = lax.axis_index("c")
    s_ax = lax.axis_index("s")
    wid = s_ax * 2 + c
    pltpu.sync_copy(x_hbm.at[wid], idx_v)
    lane = lax.iota(jnp.int32, 16)
    base_u = wid * PER_W

    def gather_start(t, rbuf, sem):
        pltpu.make_async_copy(table_hbm.at[idx_v.at[t]], rbuf, sem).start()

    def gather_wait(rbuf, sem):
        pltpu.make_async_copy(table_hbm.at[idx_v.at[0]], rbuf, sem).wait()

    hsplit = []
    for g4 in range(4):
        hv = lane + 16 * g4
        hsplit.append((hv // 8, hv % 8))

    def transpose(rbuf, tbuf):
        @plsc.parallel_loop(0, 128, unroll=4)
        def r_body(r):
            bvec = jnp.full((16,), r, jnp.int32)
            for g4 in range(4):
                v = rbuf[r, pl.ds(16 * g4, 16)]
                plsc.store_scatter(tbuf, [hsplit[g4][0], hsplit[g4][1], bvec], v)

    def out_ref(t):
        u = base_u + t
        return out_hbm.at[u // NBG, :, u % NBG]

    def out_start(t, tbuf, sem):
        pltpu.make_async_copy(tbuf.at[:, :, 0:128], out_ref(t), sem).start()

    def out_wait(tbuf, sem):
        pltpu.make_async_copy(tbuf.at[:, :, 0:128], out_ref(0), sem).wait()

    for i in range(NB):
        gather_start(i, rows[i], gsems[i])

    def body(tb, carry):
        t0 = NB * tb
        for i in range(NB):
            gather_wait(rows[i], gsems[i])

            @pl.when(tb > 0)
            def _(i=i):
                out_wait(trs[i], osems[i])

            transpose(rows[i], trs[i])

            @pl.when(tb < PER_W // NB - 1)
            def _(i=i, t0=t0):
                gather_start(t0 + i + NB, rows[i], gsems[i])

            out_start(t0 + i, trs[i], osems[i])
        return carry

    lax.fori_loop(0, PER_W // NB, body, 0)
    for i in range(NB):
        out_wait(trs[i], osems[i])


@jax.jit
def kernel(x, x_lengths, emb_weight):
    del x_lengths
    table = _scale_table(emb_weight)
    xw = (x.T.astype(jnp.int32) * 2).reshape(NW, PER_W, 128)

    mesh = plsc.VectorSubcoreMesh(core_axis_name="c", subcore_axis_name="s")
    gather = functools.partial(
        pl.kernel,
        mesh=mesh,
        out_type=jax.ShapeDtypeStruct((SEQ, 8, NBG, 8, 128), jnp.float32),
        scratch_types=(
            [pltpu.VMEM((PER_W, 128), jnp.int32)]
            + [pltpu.VMEM((128, HIDDEN), jnp.float32) for _ in range(NB)]
            + [pltpu.VMEM((8, 8, 129), jnp.float32) for _ in range(NB)]
            + [pltpu.SemaphoreType.DMA for _ in range(2 * NB)]
        ),
        compiler_params=pltpu.CompilerParams(
            use_tc_tiling_on_sc=False, needs_layout_passes=False
        ),
    )(_gather_kernel)
    out5 = gather(xw, table)
    return out5.transpose((2, 4, 0, 1, 3)).reshape(BATCH, SEQ, HIDDEN)


# speedup vs baseline: 1.1783x; 1.1783x over previous
import functools
import math

import jax
import jax.numpy as jnp
from jax import lax
from jax.experimental import pallas as pl
from jax.experimental.pallas import tpu as pltpu
from jax.experimental.pallas import tpu_sc as plsc

N_VOCAB = 100000
HIDDEN = 64
BATCH = 4096
SEQ = 200
SCALE = math.sqrt(HIDDEN)

NW = 32
NBG = BATCH // 128
NUNIT = SEQ * NBG
PER_W = NUNIT // NW
TBLK = 4096


def _scale_body(t_ref, o_ref):
    o_ref[:, 0:HIDDEN] = t_ref[...].T * SCALE


def _scale_table(emb_weight):
    embT = emb_weight.T
    scaled = pl.pallas_call(
        _scale_body,
        grid=((N_VOCAB + TBLK - 1) // TBLK,),
        in_specs=[pl.BlockSpec((HIDDEN, TBLK), lambda i: (0, i))],
        out_specs=pl.BlockSpec((TBLK, 2 * HIDDEN), lambda i: (i, 0)),
        out_shape=jax.ShapeDtypeStruct((N_VOCAB, 2 * HIDDEN), jnp.float32),
    )(embT)
    return scaled.reshape(2 * N_VOCAB, HIDDEN)


NB = 4


def _gather_kernel(x_hbm, table_hbm, out_hbm, idx_v, *bufs):
    rows = bufs[0:NB]
    trs = bufs[NB:2 * NB]
    gsems = bufs[2 * NB:3 * NB]
    osems = bufs[3 * NB:4 * NB]
    c = lax.axis_index("c")
    s_ax = lax.axis_index("s")
    wid = s_ax * 2 + c
    pltpu.sync_copy(x_hbm.at[wid], idx_v)
    lane = lax.iota(jnp.int32, 16)
    base_u = wid * PER_W

    def gather_start(t, rbuf, sem):
        pltpu.make_async_copy(table_hbm.at[idx_v.at[t]], rbuf, sem).start()

    def gather_wait(rbuf, sem):
        pltpu.make_async_copy(table_hbm.at[idx_v.at[0]], rbuf, sem).wait()

    hsplit = []
    for g4 in range(4):
        hv = lane + 16 * g4
        hsplit.append((hv // 8, hv % 8))

    def transpose(rbuf, tbuf):
        @plsc.parallel_loop(0, 128, unroll=4)
        def r_body(r):
            bvec = jnp.full((16,), r, jnp.int32)
            for g4 in range(4):
                v = rbuf[r, pl.ds(16 * g4, 16)]
                plsc.store_scatter(tbuf, [hsplit[g4][0], hsplit[g4][1], bvec], v)

    def out_ref(t):
        u = base_u + t
        return out_hbm.at[u // NBG, :, u % NBG]

    def out_start(t, tbuf, sem):
        pltpu.make_async_copy(tbuf.at[:, :, 0:128], out_ref(t), sem).start()

    def out_wait(tbuf, sem):
        pltpu.make_async_copy(tbuf.at[:, :, 0:128], out_ref(0), sem).wait()

    for i in range(NB):
        gather_start(i, rows[i], gsems[i])

    def body(tb, carry):
        t0 = NB * tb
        for i in range(NB):
            gather_wait(rows[i], gsems[i])

            @pl.when(tb > 0)
            def _(i=i):
                out_wait(trs[i], osems[i])

            transpose(rows[i], trs[i])

            @pl.when(tb < PER_W // NB - 1)
            def _(i=i, t0=t0):
                gather_start(t0 + i + NB, rows[i], gsems[i])

            out_start(t0 + i, trs[i], osems[i])
        return carry

    lax.fori_loop(0, PER_W // NB, body, 0)
    for i in range(NB):
        out_wait(trs[i], osems[i])


@jax.jit
def kernel(x, x_lengths, emb_weight):
    del x_lengths
    table = _scale_table(emb_weight)
    xw = (x.T.astype(jnp.int32) * 2).reshape(NW, PER_W, 128)

    mesh = plsc.VectorSubcoreMesh(core_axis_name="c", subcore_axis_name="s")
    gather = functools.partial(
        pl.kernel,
        mesh=mesh,
        out_type=jax.ShapeDtypeStruct((SEQ, 8, NBG, 8, 128), jnp.float32),
        scratch_types=(
            [pltpu.VMEM((PER_W, 128), jnp.int32)]
            + [pltpu.VMEM((128, HIDDEN), jnp.float32) for _ in range(NB)]
            + [pltpu.VMEM((8, 8, 129), jnp.float32) for _ in range(NB)]
            + [pltpu.SemaphoreType.DMA for _ in range(2 * NB)]
        ),
        compiler_params=pltpu.CompilerParams(
            use_tc_tiling_on_sc=False, needs_layout_passes=False
        ),
    )(_gather_kernel)
    out5 = gather(xw, table)
    return out5.transpose((2, 4, 0, 1, 3)).reshape(BATCH, SEQ, HIDDEN)


# speedup vs baseline: 1.2248x; 1.0395x over previous
import functools
import math

import jax
import jax.numpy as jnp
from jax import lax
from jax.experimental import pallas as pl
from jax.experimental.pallas import tpu as pltpu
from jax.experimental.pallas import tpu_sc as plsc

N_VOCAB = 100000
HIDDEN = 64
BATCH = 4096
SEQ = 200
SCALE = math.sqrt(HIDDEN)

NW = 32
NBG = BATCH // 128
NUNIT = SEQ * NBG
PER_W = NUNIT // NW
TBLK = 8192


def _scale_body(t_ref, o_ref):
    o_ref[:, 0:HIDDEN] = t_ref[...].T * SCALE


def _scale_table(emb_weight):
    embT = emb_weight.T
    scaled = pl.pallas_call(
        _scale_body,
        grid=((N_VOCAB + TBLK - 1) // TBLK,),
        in_specs=[pl.BlockSpec((HIDDEN, TBLK), lambda i: (0, i))],
        out_specs=pl.BlockSpec((TBLK, 2 * HIDDEN), lambda i: (i, 0)),
        out_shape=jax.ShapeDtypeStruct((N_VOCAB, 2 * HIDDEN), jnp.float32),
    )(embT)
    return scaled.reshape(2 * N_VOCAB, HIDDEN)


NB = 4


def _gather_kernel(x_hbm, table_hbm, out_hbm, idx_v, *bufs):
    rows = bufs[0:NB]
    trs = bufs[NB:2 * NB]
    gsems = bufs[2 * NB:3 * NB]
    osems = bufs[3 * NB:4 * NB]
    c = lax.axis_index("c")
    s_ax = lax.axis_index("s")
    wid = s_ax * 2 + c
    pltpu.sync_copy(x_hbm.at[wid], idx_v)
    lane = lax.iota(jnp.int32, 16)
    base_u = wid * PER_W

    def gather_start(t, rbuf, sem):
        pltpu.make_async_copy(table_hbm.at[idx_v.at[t]], rbuf, sem).start()

    def gather_wait(rbuf, sem):
        pltpu.make_async_copy(table_hbm.at[idx_v.at[0]], rbuf, sem).wait()

    hsplit = []
    for g4 in range(4):
        hv = lane + 16 * g4
        hsplit.append((hv // 8, hv % 8))

    def transpose(rbuf, tbuf):
        @plsc.parallel_loop(0, 128, unroll=4)
        def r_body(r):
            bvec = jnp.full((16,), r, jnp.int32)
            for g4 in range(4):
                v = rbuf[r, pl.ds(16 * g4, 16)]
                plsc.store_scatter(tbuf, [hsplit[g4][0], hsplit[g4][1], bvec], v)

    def out_ref(t):
        u = base_u + t
        return out_hbm.at[u // NBG, :, u % NBG]

    def out_start(t, tbuf, sem):
        pltpu.make_async_copy(tbuf.at[:, :, 0:128], out_ref(t), sem).start()

    def out_wait(tbuf, sem):
        pltpu.make_async_copy(tbuf.at[:, :, 0:128], out_ref(0), sem).wait()

    for i in range(NB):
        gather_start(i, rows[i], gsems[i])

    def body(tb, carry):
        t0 = NB * tb
        for i in range(NB):
            gather_wait(rows[i], gsems[i])

            @pl.when(tb > 0)
            def _(i=i):
                out_wait(trs[i], osems[i])

            transpose(rows[i], trs[i])

            @pl.when(tb < PER_W // NB - 1)
            def _(i=i, t0=t0):
                gather_start(t0 + i + NB, rows[i], gsems[i])

            out_start(t0 + i, trs[i], osems[i])
        return carry

    lax.fori_loop(0, PER_W // NB, body, 0)
    for i in range(NB):
        out_wait(trs[i], osems[i])


@jax.jit
def kernel(x, x_lengths, emb_weight):
    del x_lengths
    table = _scale_table(emb_weight)
    xw = (x.T.astype(jnp.int32) * 2).reshape(NW, PER_W, 128)

    mesh = plsc.VectorSubcoreMesh(core_axis_name="c", subcore_axis_name="s")
    gather = functools.partial(
        pl.kernel,
        mesh=mesh,
        out_type=jax.ShapeDtypeStruct((SEQ, 8, NBG, 8, 128), jnp.float32),
        scratch_types=(
            [pltpu.VMEM((PER_W, 128), jnp.int32)]
            + [pltpu.VMEM((128, HIDDEN), jnp.float32) for _ in range(NB)]
            + [pltpu.VMEM((8, 8, 129), jnp.float32) for _ in range(NB)]
            + [pltpu.SemaphoreType.DMA for _ in range(2 * NB)]
        ),
        compiler_params=pltpu.CompilerParams(
            use_tc_tiling_on_sc=False, needs_layout_passes=False
        ),
    )(_gather_kernel)
    out5 = gather(xw, table)
    return out5.transpose((2, 4, 0, 1, 3)).reshape(BATCH, SEQ, HIDDEN)
